# Initial kernel scaffold; baseline (speedup 1.0000x reference)
import jax
import jax.numpy as jnp
from jax.experimental import pallas as pl


def kernel(x, edge_index, batch, Wr0, br0, Wo0, Wr1, br1, Wo1, Wr2, br2, Wo2, Wr3, br3, Wo3, Wr4, br4, Wo4, Wr5, br5, Wo5, Wd, bd, Wm, bm):
    raise NotImplementedError("write your pallas kernel here")


# speedup vs baseline: 4.8386x; 4.8386x over previous
import functools

import jax
import jax.numpy as jnp
from jax import lax
from jax.experimental import pallas as pl
from jax.experimental.pallas import tpu as pltpu
from jax.experimental.pallas import tpu_sc as plsc

N = 10000
E = 320000
F = 128
H = 64
G = 64

NC = 2
NS = 16
NW = NC * NS
EP = E // NW
K = 80
ITERS = EP // K
NP = 10240
RP = NP // NS

_sc_mesh = plsc.VectorSubcoreMesh(core_axis_name="c", subcore_axis_name="s")


@functools.partial(
    pl.kernel,
    out_type=jax.ShapeDtypeStruct((NC, NP, H), jnp.float32),
    mesh=_sc_mesh,
    scratch_types=[
        pltpu.VMEM((K,), jnp.int32),
        pltpu.VMEM((K,), jnp.int32),
        pltpu.VMEM((K, H), jnp.float32),
        pltpu.VMEM_SHARED((NP, H), jnp.float32),
        pltpu.SemaphoreType.DMA,
    ],
    compiler_params=pltpu.CompilerParams(use_tc_tiling_on_sc=False),
)
def _segsum_sc(a_hbm, src_hbm, dst_hbm, zero_hbm, out_hbm,
               sidx, didx, rows, acc, sem):
    c = lax.axis_index("c")
    s = lax.axis_index("s")
    wid = c * NS + s
    pltpu.sync_copy(zero_hbm, acc.at[pl.ds(s * RP, RP)])
    plsc.subcore_barrier()

    def body(j, carry):
        base = pl.multiple_of(wid * EP + j * K, 8)
        pltpu.sync_copy(src_hbm.at[pl.ds(base, K)], sidx)
        pltpu.sync_copy(dst_hbm.at[pl.ds(base, K)], didx)
        pltpu.async_copy(a_hbm.at[sidx], rows, sem).wait()
        pltpu.sync_copy(rows, acc.at[didx], add=True)
        return carry

    lax.fori_loop(0, ITERS, body, 0)
    plsc.subcore_barrier()
    pltpu.sync_copy(acc.at[pl.ds(s * RP, RP)],
                    out_hbm.at[c, pl.ds(s * RP, RP)])


def _relu(v):
    return jnp.maximum(v, 0.0)


def _tc_first(x_ref, wr_ref, br_ref, wo_ref, a_ref, b_ref):
    h = x_ref[...]
    a_ref[...] = jnp.dot(h, wr_ref[...], preferred_element_type=jnp.float32)
    b_ref[...] = (jnp.dot(h, wo_ref[...], preferred_element_type=jnp.float32)
                  + br_ref[...])


def _tc_mid(p_ref, bprev_ref, wr_ref, br_ref, wo_ref, a_ref, b_ref):
    p = p_ref[...]
    h = _relu(p[0, :N] + p[1, :N] + bprev_ref[...])
    a_ref[...] = jnp.dot(h, wr_ref[...], preferred_element_type=jnp.float32)
    b_ref[...] = (jnp.dot(h, wo_ref[...], preferred_element_type=jnp.float32)
                  + br_ref[...])


def _tc_final(p_ref, bprev_ref, batch_ref, wd_ref, bd_ref, wm_ref, bm_ref,
              mu_ref):
    p = p_ref[...]
    h = _relu(p[0, :N] + p[1, :N] + bprev_ref[...])
    gids = lax.broadcasted_iota(jnp.int32, (N, G), 1)
    m = (batch_ref[...] == gids).astype(jnp.float32)
    dn = (((0,), (0,)), ((), ()))
    psum = lax.dot_general(m, h, dn, preferred_element_type=jnp.float32)
    ones = jnp.ones((N, 1), dtype=jnp.float32)
    cnt = lax.dot_general(m, ones, dn, preferred_element_type=jnp.float32)
    pooled = psum / jnp.maximum(cnt, 1.0)
    d = _relu(jnp.dot(pooled, wd_ref[...],
                      preferred_element_type=jnp.float32) + bd_ref[...])
    mu_ref[...] = (jnp.dot(d, wm_ref[...],
                           preferred_element_type=jnp.float32) + bm_ref[...])


_first_call = pl.pallas_call(
    _tc_first,
    out_shape=[jax.ShapeDtypeStruct((N, H), jnp.float32),
               jax.ShapeDtypeStruct((N, H), jnp.float32)],
)

_mid_call = pl.pallas_call(
    _tc_mid,
    out_shape=[jax.ShapeDtypeStruct((N, H), jnp.float32),
               jax.ShapeDtypeStruct((N, H), jnp.float32)],
)

_final_call = pl.pallas_call(
    _tc_final,
    out_shape=jax.ShapeDtypeStruct((G, 1), jnp.float32),
)


def kernel(x, edge_index, batch,
           Wr0, br0, Wo0, Wr1, br1, Wo1, Wr2, br2, Wo2,
           Wr3, br3, Wo3, Wr4, br4, Wo4, Wr5, br5, Wo5,
           Wd, bd, Wm, bm):
    src = edge_index[0]
    dst = edge_index[1]
    zeros = jnp.zeros((RP, H), dtype=jnp.float32)
    batch2d = batch.reshape(N, 1)

    params = [(Wr0, br0, Wo0), (Wr1, br1, Wo1), (Wr2, br2, Wo2),
              (Wr3, br3, Wo3), (Wr4, br4, Wo4), (Wr5, br5, Wo5)]

    a, b = _first_call(x, Wr0, br0.reshape(1, H), Wo0)
    for Wr, br, Wo in params[1:]:
        p = _segsum_sc(a, src, dst, zeros)
        a, b = _mid_call(p, b, Wr, br.reshape(1, H), Wo)
    p = _segsum_sc(a, src, dst, zeros)
    mu = _final_call(p, b, batch2d, Wd, bd.reshape(1, H),
                     Wm, bm.reshape(1, 1))
    return mu.reshape(G)
